# baseline (speedup 1.0000x reference)
import jax
import jax.numpy as jnp
from jax import lax
from jax.experimental import pallas as pl
from jax.experimental.pallas import tpu as pltpu
from jax.experimental.pallas import tpu_sc as plsc

N = 10000
NP = 10240
E = 320000
EP = 327680
NC = 2
NS = 16
HALF = NP // 2
RT = HALF // NS
ACC_R = HALF + 128
DUMP = HALF
D1 = 128
EB = 128
TILE_E = EP // NS
N_CHUNK = TILE_E // EB
CPS = 8
N_SLAB = N_CHUNK // CPS
N_ROWS2D = EP // EB
W_CHUNK = N_ROWS2D // (NC * NS)
W_SLAB = 16
N_WSLAB = W_CHUNK // W_SLAB
DROWS = NP // NS
EPS = 1e-5

_MESH = plsc.VectorSubcoreMesh(core_axis_name="c", subcore_axis_name="s")


def _deg_body(dst_hbm, zeros_hbm, ones_hbm, deg0_hbm, deg1_hbm,
              acc, ones_v, slab0, slab1, isem0, isem1, ssem):
    cid = lax.axis_index("c")
    tid = lax.axis_index("s")
    r0 = tid * DROWS
    w0 = (tid * NC + cid) * W_CHUNK
    slabs = (slab0, slab1)
    isem = (isem0, isem1)
    pltpu.sync_copy(zeros_hbm.at[pl.ds(r0, DROWS)], acc.at[pl.ds(r0, DROWS)])
    pltpu.sync_copy(ones_hbm, ones_v)
    pltpu.sync_copy(dst_hbm.at[pl.ds(w0, W_SLAB)], slab0)
    plsc.subcore_barrier()

    for g in range(N_WSLAB):
        s = slabs[g % 2]
        if g + 1 < N_WSLAB:
            pltpu.async_copy(
                dst_hbm.at[pl.ds(w0 + (g + 1) * W_SLAB, W_SLAB)],
                slabs[(g + 1) % 2], isem[(g + 1) % 2])
        for c in range(W_SLAB):
            pltpu.async_copy(ones_v, acc.at[s.at[c]], ssem, add=True)
        for c in range(W_SLAB):
            pltpu.make_async_copy(ones_v, acc.at[s.at[c]], ssem).wait()
        if g + 1 < N_WSLAB:
            pltpu.make_async_copy(
                dst_hbm.at[pl.ds(w0 + (g + 1) * W_SLAB, W_SLAB)],
                slabs[(g + 1) % 2], isem[(g + 1) % 2]).wait()
    plsc.subcore_barrier()

    @pl.when(cid == 0)
    def _():
        pltpu.sync_copy(acc.at[pl.ds(r0, DROWS)],
                        deg0_hbm.at[pl.ds(r0, DROWS)])

    @pl.when(cid == 1)
    def _():
        pltpu.sync_copy(acc.at[pl.ds(r0, DROWS)],
                        deg1_hbm.at[pl.ds(r0, DROWS)])


def _count_degrees(dst_p, zeros16, ones16):
    f = pl.kernel(
        _deg_body,
        out_type=(jax.ShapeDtypeStruct((NP, 16), jnp.float32),
                  jax.ShapeDtypeStruct((NP, 16), jnp.float32)),
        mesh=_MESH,
        scratch_types=[
            pltpu.VMEM_SHARED((NP, 16), jnp.float32),
            pltpu.VMEM((EB, 16), jnp.float32),
            pltpu.VMEM((W_SLAB, EB), jnp.int32),
            pltpu.VMEM((W_SLAB, EB), jnp.int32),
            pltpu.SemaphoreType.DMA,
            pltpu.SemaphoreType.DMA,
            pltpu.SemaphoreType.DMA,
        ],
        name="gcn_degree_count",
    )
    return f(dst_p, zeros16, ones16)


def _agg_body(tbl_hbm, src_hbm, dst_hbm, out_hbm,
              acc, stage, rows0, rows1,
              sidx0, sidx1, didx0, didx1, d2_0, d2_1,
              gsem0, gsem1):
    cid = lax.axis_index("c")
    tid = lax.axis_index("s")
    base = cid * HALF
    r0 = tid * RT
    e0 = tid * TILE_E
    rows = (rows0, rows1)
    sidx = (sidx0, sidx1)
    didx = (didx0, didx1)
    d2 = (d2_0, d2_1)
    gsem = (gsem0, gsem1)

    pltpu.sync_copy(tbl_hbm.at[pl.ds(base + r0, RT)], stage)
    pltpu.sync_copy(stage, acc.at[pl.ds(r0, RT)])

    for b in range(2):
        off = e0 + b * EB
        pltpu.sync_copy(src_hbm.at[pl.ds(off, EB)], sidx[b])
        pltpu.sync_copy(dst_hbm.at[pl.ds(off, EB)], didx[b])
        pltpu.async_copy(tbl_hbm.at[sidx[b]], rows[b], gsem[b])
    plsc.subcore_barrier()

    def body(g, carry):
        for b in range(2):
            j = 2 * g + b
            for k in range(EB // 16):
                v = didx[b][pl.ds(k * 16, 16)] - base
                ok = (v >= 0) & (v < HALF)
                d2[b][pl.ds(k * 16, 16)] = jnp.where(ok, v, DUMP)
            pltpu.make_async_copy(tbl_hbm.at[sidx[b]], rows[b],
                                  gsem[b]).wait()
            pltpu.sync_copy(rows[b], acc.at[d2[b]], add=True)
            jn = j + 2

            @pl.when(jn < N_CHUNK)
            def _():
                off = e0 + jn * EB
                pltpu.sync_copy(src_hbm.at[pl.ds(off, EB)], sidx[b])
                pltpu.sync_copy(dst_hbm.at[pl.ds(off, EB)], didx[b])
                pltpu.async_copy(tbl_hbm.at[sidx[b]], rows[b], gsem[b])

        return carry

    lax.fori_loop(0, N_CHUNK // 2, body, 0)
    plsc.subcore_barrier()

    pltpu.sync_copy(acc.at[pl.ds(r0, RT)], stage)
    pltpu.sync_copy(stage, out_hbm.at[pl.ds(base + r0, RT)])


def _aggregate(tbl, src_f, dst_f):
    f = pl.kernel(
        _agg_body,
        out_type=jax.ShapeDtypeStruct((NP, D1), jnp.float32),
        mesh=_MESH,
        scratch_types=[
            pltpu.VMEM_SHARED((ACC_R, D1), jnp.float32),
            pltpu.VMEM((RT, D1), jnp.float32),
            pltpu.VMEM((EB, D1), jnp.float32),
            pltpu.VMEM((EB, D1), jnp.float32),
            pltpu.VMEM((EB,), jnp.int32),
            pltpu.VMEM((EB,), jnp.int32),
            pltpu.VMEM((EB,), jnp.int32),
            pltpu.VMEM((EB,), jnp.int32),
            pltpu.VMEM((EB,), jnp.int32),
            pltpu.VMEM((EB,), jnp.int32),
            pltpu.SemaphoreType.DMA,
            pltpu.SemaphoreType.DMA,
        ],
        name="gcn_aggregate",
    )
    return f(tbl, src_f, dst_f)


def _dinv(deg0_ref, deg1_ref):
    cnt = deg0_ref[:, 0:1] + deg1_ref[:, 0:1]
    return lax.rsqrt(1.0 + cnt)


def _scale_body(x_ref, deg0_ref, deg1_ref, xs_ref):
    xs_ref[...] = x_ref[...] * _dinv(deg0_ref, deg1_ref)


def _scale(xp, deg0, deg1):
    return pl.pallas_call(
        _scale_body,
        grid=(NP // DROWS,),
        in_specs=[
            pl.BlockSpec((DROWS, 128), lambda i: (i, 0)),
            pl.BlockSpec((DROWS, 16), lambda i: (i, 0)),
            pl.BlockSpec((DROWS, 16), lambda i: (i, 0)),
        ],
        out_specs=pl.BlockSpec((DROWS, 128), lambda i: (i, 0)),
        out_shape=jax.ShapeDtypeStruct((NP, 128), jnp.float32),
    )(xp, deg0, deg1)


def _ln_relu(h, g_ref, be_ref):
    mu = jnp.mean(h, axis=-1, keepdims=True)
    c = h - mu
    var = jnp.mean(c * c, axis=-1, keepdims=True)
    hn = c * lax.rsqrt(var + EPS) * g_ref[...] + be_ref[...]
    return jnp.maximum(hn, 0.0)


def _layer1_body(a_ref, deg0_ref, deg1_ref, w_ref, b_ref,
                 g_ref, be_ref, lo_ref, hi_ref):
    dinv = _dinv(deg0_ref, deg1_ref)
    t = a_ref[...] * dinv
    h = jnp.dot(t, w_ref[...], preferred_element_type=jnp.float32) + b_ref[...]
    h = _ln_relu(h, g_ref, be_ref) * dinv
    lo_ref[...] = h[:, :128]
    hi_ref[...] = h[:, 128:]


def _layer1(a1, deg0, deg1, W1, b1, g1, be1):
    return pl.pallas_call(
        _layer1_body,
        grid=(NP // DROWS,),
        in_specs=[
            pl.BlockSpec((DROWS, 128), lambda i: (i, 0)),
            pl.BlockSpec((DROWS, 16), lambda i: (i, 0)),
            pl.BlockSpec((DROWS, 16), lambda i: (i, 0)),
            pl.BlockSpec((128, 256), lambda i: (0, 0)),
            pl.BlockSpec((1, 256), lambda i: (0, 0)),
            pl.BlockSpec((1, 256), lambda i: (0, 0)),
            pl.BlockSpec((1, 256), lambda i: (0, 0)),
        ],
        out_specs=[
            pl.BlockSpec((DROWS, 128), lambda i: (i, 0)),
            pl.BlockSpec((DROWS, 128), lambda i: (i, 0)),
        ],
        out_shape=[jax.ShapeDtypeStruct((NP, 128), jnp.float32),
                   jax.ShapeDtypeStruct((NP, 128), jnp.float32)],
    )(a1, deg0, deg1, W1, b1, g1, be1)


def _layer2_body(alo_ref, ahi_ref, deg0_ref, deg1_ref, w_ref, b_ref,
                 g_ref, be_ref, wo_ref, bo_ref, y_ref):
    dinv = _dinv(deg0_ref, deg1_ref)
    h = (jnp.dot(alo_ref[...] * dinv, w_ref[0:128, :],
                 preferred_element_type=jnp.float32)
         + jnp.dot(ahi_ref[...] * dinv, w_ref[128:256, :],
                   preferred_element_type=jnp.float32)
         + b_ref[...])
    h = _ln_relu(h, g_ref, be_ref)
    y_ref[...] = jnp.dot(h, wo_ref[...],
                         preferred_element_type=jnp.float32) + bo_ref[...]


def _layer2(agg_lo, agg_hi, deg0, deg1, W2, b2, g2, be2, Wo, bo):
    return pl.pallas_call(
        _layer2_body,
        grid=(NP // DROWS,),
        in_specs=[
            pl.BlockSpec((DROWS, 128), lambda i: (i, 0)),
            pl.BlockSpec((DROWS, 128), lambda i: (i, 0)),
            pl.BlockSpec((DROWS, 16), lambda i: (i, 0)),
            pl.BlockSpec((DROWS, 16), lambda i: (i, 0)),
            pl.BlockSpec((256, 256), lambda i: (0, 0)),
            pl.BlockSpec((1, 256), lambda i: (0, 0)),
            pl.BlockSpec((1, 256), lambda i: (0, 0)),
            pl.BlockSpec((1, 256), lambda i: (0, 0)),
            pl.BlockSpec((256, 16), lambda i: (0, 0)),
            pl.BlockSpec((1, 16), lambda i: (0, 0)),
        ],
        out_specs=pl.BlockSpec((DROWS, 16), lambda i: (i, 0)),
        out_shape=jax.ShapeDtypeStruct((NP, 16), jnp.float32),
    )(agg_lo, agg_hi, deg0, deg1, W2, b2, g2, be2, Wo, bo)


def kernel(x, edge_index, W1, b1, g1, be1, W2, b2, g2, be2, Wo, bo):
    src = edge_index[0].astype(jnp.int32)
    dst = edge_index[1].astype(jnp.int32)
    src_f = jnp.concatenate([src, jnp.zeros((EP - E,), jnp.int32)])
    dst_f = jnp.concatenate([dst, jnp.full((EP - E,), N, jnp.int32)])
    dst2d = dst_f.reshape(N_ROWS2D, EB)
    xp = jnp.pad(x, ((0, NP - N), (0, 0)))

    zeros16 = jnp.zeros((NP, 16), jnp.float32)
    ones16 = jnp.concatenate(
        [jnp.ones((EB, 1), jnp.float32),
         jnp.zeros((EB, 15), jnp.float32)], axis=1)

    deg0, deg1 = _count_degrees(dst2d, zeros16, ones16)

    xs = _scale(xp, deg0, deg1)
    a1 = _aggregate(xs, src_f, dst_f)

    hs_lo, hs_hi = _layer1(a1, deg0, deg1, W1,
                           b1.reshape(1, -1), g1.reshape(1, -1),
                           be1.reshape(1, -1))
    a2_lo = _aggregate(hs_lo, src_f, dst_f)
    a2_hi = _aggregate(hs_hi, src_f, dst_f)

    y = _layer2(a2_lo, a2_hi, deg0, deg1, W2,
                b2.reshape(1, -1), g2.reshape(1, -1), be2.reshape(1, -1),
                Wo, bo.reshape(1, -1))
    return y[:N]


# speedup vs baseline: 1.4425x; 1.4425x over previous
import jax
import jax.numpy as jnp
from jax import lax
from jax.experimental import pallas as pl
from jax.experimental.pallas import tpu as pltpu
from jax.experimental.pallas import tpu_sc as plsc

N = 10000
NP = 10240
E = 320000
EP = 323584
NC = 2
NS = 16
HALF = NP // 2
RT = HALF // NS
ACC_R = HALF + 128
DUMP = HALF
D1 = 128
EB = 128
TILE_E = EP // NS
N_CHUNK = TILE_E // EB
EB_W = 64
WORK_E = EP // (NC * NS)
N_WCHUNK = WORK_E // EB_W
DROWS = NP // NS
EPS = 1e-5

_MESH = plsc.VectorSubcoreMesh(core_axis_name="c", subcore_axis_name="s")


def _deg_body(dst_hbm, zeros_hbm, ones_hbm, deg0_hbm, deg1_hbm,
              acc, stage, ones_v, idx_v):
    cid = lax.axis_index("c")
    tid = lax.axis_index("s")
    r0 = tid * DROWS
    pltpu.sync_copy(zeros_hbm.at[pl.ds(r0, DROWS)], stage)
    pltpu.sync_copy(stage, acc.at[pl.ds(r0, DROWS)])
    pltpu.sync_copy(ones_hbm, ones_v)
    plsc.subcore_barrier()

    e0 = (tid * NC + cid) * WORK_E

    def body(j, carry):
        pltpu.sync_copy(dst_hbm.at[pl.ds(e0 + j * EB_W, EB_W)], idx_v)
        pltpu.sync_copy(ones_v, acc.at[idx_v], add=True)
        return carry

    lax.fori_loop(0, N_WCHUNK, body, 0)
    plsc.subcore_barrier()

    pltpu.sync_copy(acc.at[pl.ds(r0, DROWS)], stage)

    @pl.when(cid == 0)
    def _():
        pltpu.sync_copy(stage, deg0_hbm.at[pl.ds(r0, DROWS)])

    @pl.when(cid == 1)
    def _():
        pltpu.sync_copy(stage, deg1_hbm.at[pl.ds(r0, DROWS)])


def _count_degrees(dst_p, zeros16, ones16):
    f = pl.kernel(
        _deg_body,
        out_type=(jax.ShapeDtypeStruct((NP, 16), jnp.float32),
                  jax.ShapeDtypeStruct((NP, 16), jnp.float32)),
        mesh=_MESH,
        scratch_types=[
            pltpu.VMEM_SHARED((NP, 16), jnp.float32),
            pltpu.VMEM((DROWS, 16), jnp.float32),
            pltpu.VMEM((EB_W, 16), jnp.float32),
            pltpu.VMEM((EB_W,), jnp.int32),
        ],
        name="gcn_degree_count",
    )
    return f(dst_p, zeros16, ones16)


def _agg_body(tbl_hbm, src_hbm, dst_hbm, out_hbm,
              acc, stage, rows0, rows1,
              sidx0, sidx1, didx0, didx1, d2_0, d2_1,
              gsem0, gsem1):
    cid = lax.axis_index("c")
    tid = lax.axis_index("s")
    base = cid * HALF
    r0 = tid * RT
    e0 = tid * TILE_E
    rows = (rows0, rows1)
    sidx = (sidx0, sidx1)
    didx = (didx0, didx1)
    d2 = (d2_0, d2_1)
    gsem = (gsem0, gsem1)

    pltpu.sync_copy(tbl_hbm.at[pl.ds(base + r0, RT)], stage)
    pltpu.sync_copy(stage, acc.at[pl.ds(r0, RT)])

    for b in range(2):
        off = e0 + b * EB
        pltpu.sync_copy(src_hbm.at[pl.ds(off, EB)], sidx[b])
        pltpu.sync_copy(dst_hbm.at[pl.ds(off, EB)], didx[b])
        pltpu.async_copy(tbl_hbm.at[sidx[b]], rows[b], gsem[b])
    plsc.subcore_barrier()

    def body(g, carry):
        for b in range(2):
            j = 2 * g + b
            for k in range(EB // 16):
                v = didx[b][pl.ds(k * 16, 16)] - base
                ok = (v >= 0) & (v < HALF)
                d2[b][pl.ds(k * 16, 16)] = jnp.where(ok, v, DUMP)
            pltpu.make_async_copy(tbl_hbm.at[sidx[b]], rows[b],
                                  gsem[b]).wait()
            pltpu.sync_copy(rows[b], acc.at[d2[b]], add=True)
            jn = j + 2

            @pl.when(jn < N_CHUNK)
            def _():
                off = e0 + jn * EB
                pltpu.sync_copy(src_hbm.at[pl.ds(off, EB)], sidx[b])
                pltpu.sync_copy(dst_hbm.at[pl.ds(off, EB)], didx[b])
                pltpu.async_copy(tbl_hbm.at[sidx[b]], rows[b], gsem[b])

        return carry

    lax.fori_loop(0, N_CHUNK // 2, body, 0)
    plsc.subcore_barrier()

    pltpu.sync_copy(acc.at[pl.ds(r0, RT)], stage)
    pltpu.sync_copy(stage, out_hbm.at[pl.ds(base + r0, RT)])


def _aggregate(tbl, src_f, dst_f):
    f = pl.kernel(
        _agg_body,
        out_type=jax.ShapeDtypeStruct((NP, D1), jnp.float32),
        mesh=_MESH,
        scratch_types=[
            pltpu.VMEM_SHARED((ACC_R, D1), jnp.float32),
            pltpu.VMEM((RT, D1), jnp.float32),
            pltpu.VMEM((EB, D1), jnp.float32),
            pltpu.VMEM((EB, D1), jnp.float32),
            pltpu.VMEM((EB,), jnp.int32),
            pltpu.VMEM((EB,), jnp.int32),
            pltpu.VMEM((EB,), jnp.int32),
            pltpu.VMEM((EB,), jnp.int32),
            pltpu.VMEM((EB,), jnp.int32),
            pltpu.VMEM((EB,), jnp.int32),
            pltpu.SemaphoreType.DMA,
            pltpu.SemaphoreType.DMA,
        ],
        name="gcn_aggregate",
    )
    return f(tbl, src_f, dst_f)


def _dinv(deg0_ref, deg1_ref):
    cnt = deg0_ref[:, 0:1] + deg1_ref[:, 0:1]
    return lax.rsqrt(1.0 + cnt)


def _scale_body(x_ref, deg0_ref, deg1_ref, xs_ref):
    xs_ref[...] = x_ref[...] * _dinv(deg0_ref, deg1_ref)


def _scale(xp, deg0, deg1):
    return pl.pallas_call(
        _scale_body,
        grid=(NP // DROWS,),
        in_specs=[
            pl.BlockSpec((DROWS, 128), lambda i: (i, 0)),
            pl.BlockSpec((DROWS, 16), lambda i: (i, 0)),
            pl.BlockSpec((DROWS, 16), lambda i: (i, 0)),
        ],
        out_specs=pl.BlockSpec((DROWS, 128), lambda i: (i, 0)),
        out_shape=jax.ShapeDtypeStruct((NP, 128), jnp.float32),
    )(xp, deg0, deg1)


def _ln_relu(h, g_ref, be_ref):
    mu = jnp.mean(h, axis=-1, keepdims=True)
    c = h - mu
    var = jnp.mean(c * c, axis=-1, keepdims=True)
    hn = c * lax.rsqrt(var + EPS) * g_ref[...] + be_ref[...]
    return jnp.maximum(hn, 0.0)


def _layer1_body(a_ref, deg0_ref, deg1_ref, w_ref, b_ref,
                 g_ref, be_ref, lo_ref, hi_ref):
    dinv = _dinv(deg0_ref, deg1_ref)
    t = a_ref[...] * dinv
    h = jnp.dot(t, w_ref[...], preferred_element_type=jnp.float32) + b_ref[...]
    h = _ln_relu(h, g_ref, be_ref) * dinv
    lo_ref[...] = h[:, :128]
    hi_ref[...] = h[:, 128:]


def _layer1(a1, deg0, deg1, W1, b1, g1, be1):
    return pl.pallas_call(
        _layer1_body,
        grid=(NP // DROWS,),
        in_specs=[
            pl.BlockSpec((DROWS, 128), lambda i: (i, 0)),
            pl.BlockSpec((DROWS, 16), lambda i: (i, 0)),
            pl.BlockSpec((DROWS, 16), lambda i: (i, 0)),
            pl.BlockSpec((128, 256), lambda i: (0, 0)),
            pl.BlockSpec((1, 256), lambda i: (0, 0)),
            pl.BlockSpec((1, 256), lambda i: (0, 0)),
            pl.BlockSpec((1, 256), lambda i: (0, 0)),
        ],
        out_specs=[
            pl.BlockSpec((DROWS, 128), lambda i: (i, 0)),
            pl.BlockSpec((DROWS, 128), lambda i: (i, 0)),
        ],
        out_shape=[jax.ShapeDtypeStruct((NP, 128), jnp.float32),
                   jax.ShapeDtypeStruct((NP, 128), jnp.float32)],
    )(a1, deg0, deg1, W1, b1, g1, be1)


def _layer2_body(alo_ref, ahi_ref, deg0_ref, deg1_ref, w_ref, b_ref,
                 g_ref, be_ref, wo_ref, bo_ref, y_ref):
    dinv = _dinv(deg0_ref, deg1_ref)
    h = (jnp.dot(alo_ref[...] * dinv, w_ref[0:128, :],
                 preferred_element_type=jnp.float32)
         + jnp.dot(ahi_ref[...] * dinv, w_ref[128:256, :],
                   preferred_element_type=jnp.float32)
         + b_ref[...])
    h = _ln_relu(h, g_ref, be_ref)
    y_ref[...] = jnp.dot(h, wo_ref[...],
                         preferred_element_type=jnp.float32) + bo_ref[...]


def _layer2(agg_lo, agg_hi, deg0, deg1, W2, b2, g2, be2, Wo, bo):
    return pl.pallas_call(
        _layer2_body,
        grid=(NP // DROWS,),
        in_specs=[
            pl.BlockSpec((DROWS, 128), lambda i: (i, 0)),
            pl.BlockSpec((DROWS, 128), lambda i: (i, 0)),
            pl.BlockSpec((DROWS, 16), lambda i: (i, 0)),
            pl.BlockSpec((DROWS, 16), lambda i: (i, 0)),
            pl.BlockSpec((256, 256), lambda i: (0, 0)),
            pl.BlockSpec((1, 256), lambda i: (0, 0)),
            pl.BlockSpec((1, 256), lambda i: (0, 0)),
            pl.BlockSpec((1, 256), lambda i: (0, 0)),
            pl.BlockSpec((256, 16), lambda i: (0, 0)),
            pl.BlockSpec((1, 16), lambda i: (0, 0)),
        ],
        out_specs=pl.BlockSpec((DROWS, 16), lambda i: (i, 0)),
        out_shape=jax.ShapeDtypeStruct((NP, 16), jnp.float32),
    )(agg_lo, agg_hi, deg0, deg1, W2, b2, g2, be2, Wo, bo)


def kernel(x, edge_index, W1, b1, g1, be1, W2, b2, g2, be2, Wo, bo):
    src = edge_index[0].astype(jnp.int32)
    dst = edge_index[1].astype(jnp.int32)
    src_f = jnp.concatenate([src, jnp.zeros((EP - E,), jnp.int32)])
    dst_f = jnp.concatenate([dst, jnp.full((EP - E,), N, jnp.int32)])
    xp = jnp.pad(x, ((0, NP - N), (0, 0)))

    zeros16 = jnp.zeros((NP, 16), jnp.float32)
    ones16 = jnp.concatenate(
        [jnp.ones((EB_W, 1), jnp.float32),
         jnp.zeros((EB_W, 15), jnp.float32)], axis=1)

    deg0, deg1 = _count_degrees(dst_f, zeros16, ones16)

    xs = _scale(xp, deg0, deg1)
    a1 = _aggregate(xs, src_f, dst_f)

    hs_lo, hs_hi = _layer1(a1, deg0, deg1, W1,
                           b1.reshape(1, -1), g1.reshape(1, -1),
                           be1.reshape(1, -1))
    a2_lo = _aggregate(hs_lo, src_f, dst_f)
    a2_hi = _aggregate(hs_hi, src_f, dst_f)

    y = _layer2(a2_lo, a2_hi, deg0, deg1, W2,
                b2.reshape(1, -1), g2.reshape(1, -1), be2.reshape(1, -1),
                Wo, bo.reshape(1, -1))
    return y[:N]


# speedup vs baseline: 1.4667x; 1.0168x over previous
import jax
import jax.numpy as jnp
from jax import lax
from jax.experimental import pallas as pl
from jax.experimental.pallas import tpu as pltpu
from jax.experimental.pallas import tpu_sc as plsc

N = 10000
NP = 10240
E = 320000
EP = 323584
NC = 2
NS = 16
HALF = NP // 2
RT = HALF // NS
ACC_R = HALF + 128
DUMP = HALF
D1 = 128
EB = 128
TILE_E = EP // NS
N_CHUNK = TILE_E // EB
EB_W = 64
WORK_E = EP // (NC * NS)
N_WCHUNK = WORK_E // EB_W
DROWS = NP // NS
EPS = 1e-5

_MESH = plsc.VectorSubcoreMesh(core_axis_name="c", subcore_axis_name="s")


def _deg_body(dst_hbm, zeros_hbm, ones_hbm, deg0_hbm, deg1_hbm,
              acc, stage, ones_v, idx_v):
    cid = lax.axis_index("c")
    tid = lax.axis_index("s")
    r0 = tid * DROWS
    pltpu.sync_copy(zeros_hbm.at[pl.ds(r0, DROWS)], stage)
    pltpu.sync_copy(stage, acc.at[pl.ds(r0, DROWS)])
    pltpu.sync_copy(ones_hbm, ones_v)
    plsc.subcore_barrier()

    e0 = (tid * NC + cid) * WORK_E

    def body(j, carry):
        pltpu.sync_copy(dst_hbm.at[pl.ds(e0 + j * EB_W, EB_W)], idx_v)
        pltpu.sync_copy(ones_v, acc.at[idx_v], add=True)
        return carry

    lax.fori_loop(0, N_WCHUNK, body, 0)
    plsc.subcore_barrier()

    pltpu.sync_copy(acc.at[pl.ds(r0, DROWS)], stage)

    @pl.when(cid == 0)
    def _():
        pltpu.sync_copy(stage, deg0_hbm.at[pl.ds(r0, DROWS)])

    @pl.when(cid == 1)
    def _():
        pltpu.sync_copy(stage, deg1_hbm.at[pl.ds(r0, DROWS)])


def _count_degrees(dst_p, zeros16, ones16):
    f = pl.kernel(
        _deg_body,
        out_type=(jax.ShapeDtypeStruct((NP, 16), jnp.float32),
                  jax.ShapeDtypeStruct((NP, 16), jnp.float32)),
        mesh=_MESH,
        scratch_types=[
            pltpu.VMEM_SHARED((NP, 16), jnp.float32),
            pltpu.VMEM((DROWS, 16), jnp.float32),
            pltpu.VMEM((EB_W, 16), jnp.float32),
            pltpu.VMEM((EB_W,), jnp.int32),
        ],
        name="gcn_degree_count",
    )
    return f(dst_p, zeros16, ones16)


def _agg_body(tbl_hbm, src_hbm, dst_hbm, out_hbm,
              acc, stage, rows0, rows1,
              sidx0, sidx1, didx0, didx1, d2_0, d2_1,
              gsem0, gsem1):
    cid = lax.axis_index("c")
    tid = lax.axis_index("s")
    base = cid * HALF
    r0 = tid * RT
    e0 = tid * TILE_E
    rows = (rows0, rows1)
    sidx = (sidx0, sidx1)
    didx = (didx0, didx1)
    d2 = (d2_0, d2_1)
    gsem = (gsem0, gsem1)

    pltpu.sync_copy(tbl_hbm.at[pl.ds(base + r0, RT)], stage)
    pltpu.sync_copy(stage, acc.at[pl.ds(r0, RT)])

    for b in range(2):
        off = e0 + b * EB
        pltpu.sync_copy(src_hbm.at[pl.ds(off, EB)], sidx[b])
        pltpu.sync_copy(dst_hbm.at[pl.ds(off, EB)], didx[b])
        pltpu.async_copy(tbl_hbm.at[sidx[b]], rows[b], gsem[b])
    plsc.subcore_barrier()

    def body(g, carry):
        for b in range(2):
            j = 2 * g + b
            for k in range(EB // 16):
                v = didx[b][pl.ds(k * 16, 16)] - base
                ok = (v >= 0) & (v < HALF)
                dump = lax.iota(jnp.int32, 16) + (DUMP + 16 * (k % 8))
                d2[b][pl.ds(k * 16, 16)] = jnp.where(ok, v, dump)
            pltpu.make_async_copy(tbl_hbm.at[sidx[b]], rows[b],
                                  gsem[b]).wait()
            pltpu.sync_copy(rows[b], acc.at[d2[b]], add=True)
            jn = j + 2

            @pl.when(jn < N_CHUNK)
            def _():
                off = e0 + jn * EB
                pltpu.sync_copy(src_hbm.at[pl.ds(off, EB)], sidx[b])
                pltpu.sync_copy(dst_hbm.at[pl.ds(off, EB)], didx[b])
                pltpu.async_copy(tbl_hbm.at[sidx[b]], rows[b], gsem[b])

        return carry

    lax.fori_loop(0, N_CHUNK // 2, body, 0)
    plsc.subcore_barrier()

    pltpu.sync_copy(acc.at[pl.ds(r0, RT)], stage)
    pltpu.sync_copy(stage, out_hbm.at[pl.ds(base + r0, RT)])


def _aggregate(tbl, src_f, dst_f):
    f = pl.kernel(
        _agg_body,
        out_type=jax.ShapeDtypeStruct((NP, D1), jnp.float32),
        mesh=_MESH,
        scratch_types=[
            pltpu.VMEM_SHARED((ACC_R, D1), jnp.float32),
            pltpu.VMEM((RT, D1), jnp.float32),
            pltpu.VMEM((EB, D1), jnp.float32),
            pltpu.VMEM((EB, D1), jnp.float32),
            pltpu.VMEM((EB,), jnp.int32),
            pltpu.VMEM((EB,), jnp.int32),
            pltpu.VMEM((EB,), jnp.int32),
            pltpu.VMEM((EB,), jnp.int32),
            pltpu.VMEM((EB,), jnp.int32),
            pltpu.VMEM((EB,), jnp.int32),
            pltpu.SemaphoreType.DMA,
            pltpu.SemaphoreType.DMA,
        ],
        name="gcn_aggregate",
    )
    return f(tbl, src_f, dst_f)


def _dinv(deg0_ref, deg1_ref):
    cnt = deg0_ref[:, 0:1] + deg1_ref[:, 0:1]
    return lax.rsqrt(1.0 + cnt)


def _scale_body(x_ref, deg0_ref, deg1_ref, xs_ref):
    xs_ref[...] = x_ref[...] * _dinv(deg0_ref, deg1_ref)


def _scale(xp, deg0, deg1):
    return pl.pallas_call(
        _scale_body,
        grid=(NP // DROWS,),
        in_specs=[
            pl.BlockSpec((DROWS, 128), lambda i: (i, 0)),
            pl.BlockSpec((DROWS, 16), lambda i: (i, 0)),
            pl.BlockSpec((DROWS, 16), lambda i: (i, 0)),
        ],
        out_specs=pl.BlockSpec((DROWS, 128), lambda i: (i, 0)),
        out_shape=jax.ShapeDtypeStruct((NP, 128), jnp.float32),
    )(xp, deg0, deg1)


def _ln_relu(h, g_ref, be_ref):
    mu = jnp.mean(h, axis=-1, keepdims=True)
    c = h - mu
    var = jnp.mean(c * c, axis=-1, keepdims=True)
    hn = c * lax.rsqrt(var + EPS) * g_ref[...] + be_ref[...]
    return jnp.maximum(hn, 0.0)


def _layer1_body(a_ref, deg0_ref, deg1_ref, w_ref, b_ref,
                 g_ref, be_ref, lo_ref, hi_ref):
    dinv = _dinv(deg0_ref, deg1_ref)
    t = a_ref[...] * dinv
    h = jnp.dot(t, w_ref[...], preferred_element_type=jnp.float32) + b_ref[...]
    h = _ln_relu(h, g_ref, be_ref) * dinv
    lo_ref[...] = h[:, :128]
    hi_ref[...] = h[:, 128:]


def _layer1(a1, deg0, deg1, W1, b1, g1, be1):
    return pl.pallas_call(
        _layer1_body,
        grid=(NP // DROWS,),
        in_specs=[
            pl.BlockSpec((DROWS, 128), lambda i: (i, 0)),
            pl.BlockSpec((DROWS, 16), lambda i: (i, 0)),
            pl.BlockSpec((DROWS, 16), lambda i: (i, 0)),
            pl.BlockSpec((128, 256), lambda i: (0, 0)),
            pl.BlockSpec((1, 256), lambda i: (0, 0)),
            pl.BlockSpec((1, 256), lambda i: (0, 0)),
            pl.BlockSpec((1, 256), lambda i: (0, 0)),
        ],
        out_specs=[
            pl.BlockSpec((DROWS, 128), lambda i: (i, 0)),
            pl.BlockSpec((DROWS, 128), lambda i: (i, 0)),
        ],
        out_shape=[jax.ShapeDtypeStruct((NP, 128), jnp.float32),
                   jax.ShapeDtypeStruct((NP, 128), jnp.float32)],
    )(a1, deg0, deg1, W1, b1, g1, be1)


def _layer2_body(alo_ref, ahi_ref, deg0_ref, deg1_ref, w_ref, b_ref,
                 g_ref, be_ref, wo_ref, bo_ref, y_ref):
    dinv = _dinv(deg0_ref, deg1_ref)
    h = (jnp.dot(alo_ref[...] * dinv, w_ref[0:128, :],
                 preferred_element_type=jnp.float32)
         + jnp.dot(ahi_ref[...] * dinv, w_ref[128:256, :],
                   preferred_element_type=jnp.float32)
         + b_ref[...])
    h = _ln_relu(h, g_ref, be_ref)
    y_ref[...] = jnp.dot(h, wo_ref[...],
                         preferred_element_type=jnp.float32) + bo_ref[...]


def _layer2(agg_lo, agg_hi, deg0, deg1, W2, b2, g2, be2, Wo, bo):
    return pl.pallas_call(
        _layer2_body,
        grid=(NP // DROWS,),
        in_specs=[
            pl.BlockSpec((DROWS, 128), lambda i: (i, 0)),
            pl.BlockSpec((DROWS, 128), lambda i: (i, 0)),
            pl.BlockSpec((DROWS, 16), lambda i: (i, 0)),
            pl.BlockSpec((DROWS, 16), lambda i: (i, 0)),
            pl.BlockSpec((256, 256), lambda i: (0, 0)),
            pl.BlockSpec((1, 256), lambda i: (0, 0)),
            pl.BlockSpec((1, 256), lambda i: (0, 0)),
            pl.BlockSpec((1, 256), lambda i: (0, 0)),
            pl.BlockSpec((256, 16), lambda i: (0, 0)),
            pl.BlockSpec((1, 16), lambda i: (0, 0)),
        ],
        out_specs=pl.BlockSpec((DROWS, 16), lambda i: (i, 0)),
        out_shape=jax.ShapeDtypeStruct((NP, 16), jnp.float32),
    )(agg_lo, agg_hi, deg0, deg1, W2, b2, g2, be2, Wo, bo)


def kernel(x, edge_index, W1, b1, g1, be1, W2, b2, g2, be2, Wo, bo):
    src = edge_index[0].astype(jnp.int32)
    dst = edge_index[1].astype(jnp.int32)
    src_f = jnp.concatenate([src, jnp.zeros((EP - E,), jnp.int32)])
    dst_f = jnp.concatenate(
        [dst, N + jnp.arange(EP - E, dtype=jnp.int32) % (NP - N)])
    xp = jnp.pad(x, ((0, NP - N), (0, 0)))

    zeros16 = jnp.zeros((NP, 16), jnp.float32)
    ones16 = jnp.concatenate(
        [jnp.ones((EB_W, 1), jnp.float32),
         jnp.zeros((EB_W, 15), jnp.float32)], axis=1)

    deg0, deg1 = _count_degrees(dst_f, zeros16, ones16)

    xs = _scale(xp, deg0, deg1)
    a1 = _aggregate(xs, src_f, dst_f)

    hs_lo, hs_hi = _layer1(a1, deg0, deg1, W1,
                           b1.reshape(1, -1), g1.reshape(1, -1),
                           be1.reshape(1, -1))
    a2_lo = _aggregate(hs_lo, src_f, dst_f)
    a2_hi = _aggregate(hs_hi, src_f, dst_f)

    y = _layer2(a2_lo, a2_hi, deg0, deg1, W2,
                b2.reshape(1, -1), g2.reshape(1, -1), be2.reshape(1, -1),
                Wo, bo.reshape(1, -1))
    return y[:N]
